# baseline (speedup 1.0000x reference)
import functools

import jax
import jax.numpy as jnp
from jax import lax
from jax.experimental import pallas as pl
from jax.experimental.pallas import tpu as pltpu
from jax.experimental.pallas import tpu_sc as plsc

N = 10000
E = 320000
D = 128
C = 32
H = 16
K = 4
NP = 10240
L = 16
NS = 16
NC = 2
CH = 128
NCHT = 160
TE = NCHT * CH
EP = NS * TE
EROWS = EP // CH
TSTR = NP // NS

_mesh = plsc.VectorSubcoreMesh(core_axis_name="c", subcore_axis_name="s")


def _iota16():
    return lax.iota(jnp.int32, L)


def _f16i(v):
    return jnp.full((L,), v, dtype=jnp.int32)


def _newton_rsqrt(d):
    i = plsc.bitcast(d, jnp.int32)
    y = plsc.bitcast(jnp.int32(0x5F3759DF) - lax.shift_right_arithmetic(i, 1),
                     jnp.float32)
    for _ in range(3):
        y = y * (1.5 - 0.5 * d * y * y)
    return y


TE2 = TE // 2


def _prep_body(src_in, dst_in, ea_in, norm_out,
               src_v, dst_v, ea_v, degl_v, acc_v, tmp_v, dis_v, dfull_v,
               nbuf_v, deg_sh, dis_sh):
    c = lax.axis_index("c")
    s = lax.axis_index("s")
    pltpu.sync_copy(src_in.at[pl.ds(s * TE, TE)], src_v)
    pltpu.sync_copy(dst_in.at[pl.ds(s * TE, TE)], dst_v)
    pltpu.sync_copy(ea_in.at[pl.ds(s * TE, TE)], ea_v)

    def _z(i, _):
        degl_v[pl.ds(i * L, L)] = jnp.zeros((L,), jnp.float32)
        return 0
    lax.fori_loop(0, NP // L, _z, 0)

    def _deg(i, _):
        d16 = dst_v[pl.ds(i * L, L)]
        e16 = ea_v[pl.ds(i * L, L)]
        plsc.addupdate_scatter(degl_v, [d16], e16)
        return 0
    lax.fori_loop(0, TE // L, _deg, 0)
    pltpu.sync_copy(degl_v, deg_sh.at[s])
    plsc.subcore_barrier()

    def _za(r, _):
        acc_v[pl.ds(r * L, L)] = jnp.zeros((L,), jnp.float32)
        return 0
    lax.fori_loop(0, TSTR // L, _za, 0)
    for u in range(NS):
        pltpu.sync_copy(deg_sh.at[u, pl.ds(s * TSTR, TSTR)], tmp_v)

        def _ra(r, _):
            acc_v[pl.ds(r * L, L)] = acc_v[pl.ds(r * L, L)] + \
                tmp_v[pl.ds(r * L, L)]
            return 0
        lax.fori_loop(0, TSTR // L, _ra, 0)

    def _dis(r, _):
        d = acc_v[pl.ds(r * L, L)]
        y = _newton_rsqrt(d + 1e-12)
        dis_v[pl.ds(r * L, L)] = jnp.where(d > 0.0, y, 0.0)
        return 0
    lax.fori_loop(0, TSTR // L, _dis, 0)
    pltpu.sync_copy(dis_v, dis_sh.at[pl.ds(s * TSTR, TSTR)])
    plsc.subcore_barrier()
    pltpu.sync_copy(dis_sh, dfull_v)

    def _nrm(i, _):
        o = c * TE2 + i * L
        s16 = src_v[pl.ds(o, L)]
        d16 = dst_v[pl.ds(o, L)]
        e16 = ea_v[pl.ds(o, L)]
        n16 = plsc.load_gather(dfull_v, [s16]) * e16 * \
            plsc.load_gather(dfull_v, [d16])
        nbuf_v[pl.ds(i * L, L)] = n16
        return 0
    lax.fori_loop(0, TE2 // L, _nrm, 0)
    pltpu.sync_copy(nbuf_v, norm_out.at[pl.ds(s * TE + c * TE2, TE2)])


_sc_prep = pl.kernel(
    _prep_body,
    out_type=jax.ShapeDtypeStruct((EP,), jnp.float32),
    mesh=_mesh,
    scratch_types=[
        pltpu.VMEM((TE,), jnp.int32),
        pltpu.VMEM((TE,), jnp.int32),
        pltpu.VMEM((TE,), jnp.float32),
        pltpu.VMEM((NP,), jnp.float32),
        pltpu.VMEM((TSTR,), jnp.float32),
        pltpu.VMEM((TSTR,), jnp.float32),
        pltpu.VMEM((TSTR,), jnp.float32),
        pltpu.VMEM((NP,), jnp.float32),
        pltpu.VMEM((TE2,), jnp.float32),
        pltpu.VMEM_SHARED((NS, NP), jnp.float32),
        pltpu.VMEM_SHARED((NP,), jnp.float32),
    ],
    compiler_params=pltpu.CompilerParams(
        needs_layout_passes=False, use_tc_tiling_on_sc=False),
    name="sc_prep",
)


NBUF = 4
NJO = NCHT // NBUF - 1


def _layer_body(x_in, src_in, dst_in, norm_in, to0, to1, to2, to3,
                src_v, dst_v, norm_v, g0, g1, g2, g3, b0, b1, b2, b3,
                zero_v, xa_sh, s0, s1, s2, s3, t0, t1, t2, t3):
    c = lax.axis_index("c")
    s = lax.axis_index("s")
    taps_out = (to0, to1, to2, to3)
    gbufs = (g0, g1, g2, g3)
    sbufs = (b0, b1, b2, b3)
    sems = (s0, s1, s2, s3)
    ssems = (t0, t1, t2, t3)
    r0 = s * NCHT
    pltpu.sync_copy(src_in.at[pl.ds(r0, NCHT)], src_v)
    pltpu.sync_copy(dst_in.at[pl.ds(r0, NCHT)], dst_v)
    pltpu.sync_copy(norm_in.at[pl.ds(s * TE, TE)], norm_v)

    off = c * NP

    def _o(j, _):
        for kk in range(CH // L):
            v = src_v[j, pl.ds(kk * L, L)]
            src_v[j, pl.ds(kk * L, L)] = v + off
        return 0
    lax.fori_loop(0, NCHT, _o, 0)

    def _z(i, _):
        zero_v[i] = jnp.zeros((L,), jnp.float32)
        return 0
    lax.fori_loop(0, TSTR, _z, 0)

    pltpu.sync_copy(zero_v, xa_sh.at[pl.ds(s * TSTR, TSTR)])
    plsc.subcore_barrier()

    def _scale(j, b):
        def _g(g, _):
            n16 = norm_v[pl.ds(j * CH + g * L, L)]
            for e2 in range(L):
                spl = n16.at[_f16i(e2)].get(mode="promise_in_bounds")
                r = g * L + e2
                sbufs[b][r] = gbufs[b][r] * spl
            return 0
        lax.fori_loop(0, CH // L, _g, 0)

    def _chunk(j, b, xs, ys, wait_scat, issue_next):
        pltpu.make_async_copy(xs.at[src_v.at[j]], gbufs[b], sems[b]).wait()
        if wait_scat:
            pltpu.make_async_copy(sbufs[b], ys.at[dst_v.at[j]],
                                  ssems[b]).wait()
        _scale(j, b)
        pltpu.async_copy(sbufs[b], ys.at[dst_v.at[j]], ssems[b], add=True)
        if issue_next:
            pltpu.async_copy(xs.at[src_v.at[j + NBUF]], gbufs[b], sems[b])

    ys = xa_sh
    srcs = (x_in,) + taps_out[:-1]
    for k in range(K):
        xs = srcs[k]
        for b in range(NBUF):
            pltpu.async_copy(xs.at[src_v.at[b]], gbufs[b], sems[b])
        for b in range(NBUF):
            _chunk(b, b, xs, ys, False, True)

        def _pipe(jo, _, xs=xs, ys=ys):
            for b in range(NBUF):
                _chunk(jo * NBUF + b, b, xs, ys, True, True)
            return 0
        lax.fori_loop(1, NJO, _pipe, 0)
        for b in range(NBUF):
            _chunk(NJO * NBUF + b, b, xs, ys, True, False)
        for b in range(NBUF):
            pltpu.make_async_copy(sbufs[b], ys.at[dst_v.at[b]],
                                  ssems[b]).wait()
        plsc.subcore_barrier()
        pltpu.sync_copy(ys.at[pl.ds(s * TSTR, TSTR)],
                        taps_out[k].at[pl.ds(c * NP + s * TSTR, TSTR)])
        if k < K - 1:
            pltpu.sync_copy(zero_v, ys.at[pl.ds(s * TSTR, TSTR)])
        plsc.subcore_barrier()


_sc_layer = pl.kernel(
    _layer_body,
    out_type=[jax.ShapeDtypeStruct((2 * NP, H), jnp.float32)] * K,
    mesh=_mesh,
    scratch_types=[
        pltpu.VMEM((NCHT, CH), jnp.int32),
        pltpu.VMEM((NCHT, CH), jnp.int32),
        pltpu.VMEM((TE,), jnp.float32),
        pltpu.VMEM((CH, H), jnp.float32),
        pltpu.VMEM((CH, H), jnp.float32),
        pltpu.VMEM((CH, H), jnp.float32),
        pltpu.VMEM((CH, H), jnp.float32),
        pltpu.VMEM((CH, H), jnp.float32),
        pltpu.VMEM((CH, H), jnp.float32),
        pltpu.VMEM((CH, H), jnp.float32),
        pltpu.VMEM((CH, H), jnp.float32),
        pltpu.VMEM((TSTR, H), jnp.float32),
        pltpu.VMEM_SHARED((NP, H), jnp.float32),
        pltpu.SemaphoreType.DMA,
        pltpu.SemaphoreType.DMA,
        pltpu.SemaphoreType.DMA,
        pltpu.SemaphoreType.DMA,
        pltpu.SemaphoreType.DMA,
        pltpu.SemaphoreType.DMA,
        pltpu.SemaphoreType.DMA,
        pltpu.SemaphoreType.DMA,
    ],
    compiler_params=pltpu.CompilerParams(
        needs_layout_passes=False, use_tc_tiling_on_sc=False),
    name="sc_layer",
)


_BLK = 1024
_NBLK = NP // _BLK


def _leaky(t):
    return jnp.where(t >= 0.0, t, 0.01 * t)


def _pre_body(x_ref, w_ref, b_ref, o_ref):
    t = jnp.dot(x_ref[...], w_ref[...],
                preferred_element_type=jnp.float32) + b_ref[...]
    h = _leaky(t)
    o_ref[0] = h[:, :H]
    o_ref[1] = h[:, H:]


def _tc_pre(state_p, win, b_in):
    return pl.pallas_call(
        _pre_body,
        grid=(_NBLK,),
        in_specs=[
            pl.BlockSpec((_BLK, D), lambda i: (i, 0)),
            pl.BlockSpec((D, C), lambda i: (0, 0)),
            pl.BlockSpec((1, C), lambda i: (0, 0)),
        ],
        out_specs=pl.BlockSpec((2, _BLK, H), lambda i: (0, i, 0)),
        out_shape=jax.ShapeDtypeStruct((2, NP, H), jnp.float32),
    )(state_p, win, b_in)


def _cat(t):
    return jnp.concatenate([t[0], t[1]], axis=1)


def _acc_taps(taps, w_ref, b_ref):
    acc = b_ref[...]
    for k, t in enumerate(taps):
        acc = acc + jnp.dot(_cat(t[...]), w_ref[k],
                            preferred_element_type=jnp.float32)
    return acc


def _combine_body(t0, t1, t2, t3, t4, w_ref, b_ref, o_ref):
    h = _leaky(_acc_taps((t0, t1, t2, t3, t4), w_ref, b_ref))
    o_ref[0] = h[:, :H]
    o_ref[1] = h[:, H:]


def _tc_combine(taps, w, b):
    return pl.pallas_call(
        _combine_body,
        grid=(_NBLK,),
        in_specs=[pl.BlockSpec((2, _BLK, H), lambda i: (0, i, 0))] * 5 + [
            pl.BlockSpec((K + 1, C, C), lambda i: (0, 0, 0)),
            pl.BlockSpec((1, C), lambda i: (0, 0)),
        ],
        out_specs=pl.BlockSpec((2, _BLK, H), lambda i: (0, i, 0)),
        out_shape=jax.ShapeDtypeStruct((2, NP, H), jnp.float32),
    )(*taps, w, b)


def _final_body(t0, t1, t2, t3, t4, w_ref, b_ref, wo_ref, bo_ref,
                mu_ref, sg_ref):
    h = _leaky(_acc_taps((t0, t1, t2, t3, t4), w_ref, b_ref))
    o = jnp.dot(h, wo_ref[...], preferred_element_type=jnp.float32) + \
        bo_ref[...]
    mu_ref[...] = o[:, :H]
    o2 = o[:, H:]
    sg_ref[...] = jnp.maximum(o2, 0.0) + jnp.log1p(jnp.exp(-jnp.abs(o2)))


def _tc_final(taps, w, b, wo, bo):
    return pl.pallas_call(
        _final_body,
        grid=(_NBLK,),
        in_specs=[pl.BlockSpec((2, _BLK, H), lambda i: (0, i, 0))] * 5 + [
            pl.BlockSpec((K + 1, C, C), lambda i: (0, 0, 0)),
            pl.BlockSpec((1, C), lambda i: (0, 0)),
            pl.BlockSpec((C, C), lambda i: (0, 0)),
            pl.BlockSpec((1, C), lambda i: (0, 0)),
        ],
        out_specs=[pl.BlockSpec((_BLK, H), lambda i: (i, 0))] * 2,
        out_shape=[jax.ShapeDtypeStruct((NP, H), jnp.float32)] * 2,
    )(*taps, w, b, wo, bo)


def kernel(state, edge_index, edge_attr, Win, b_in, W1, b1, W2, b2,
           Wout, bout):
    src = edge_index[0].astype(jnp.int32)
    dst = edge_index[1].astype(jnp.int32)
    pad = EP - E
    src_f = jnp.pad(src, (0, pad))
    dst_f = jnp.pad(dst, (0, pad))
    src2 = src_f.reshape(EROWS, CH)
    dst2 = dst_f.reshape(EROWS, CH)
    ea_f = jnp.pad(edge_attr, (0, pad))
    state_p = jnp.pad(state, ((0, NP - N), (0, 0)))

    norm_f = _sc_prep(src_f, dst_f, ea_f)
    x0 = _tc_pre(state_p, Win, b_in.reshape(1, C))

    t1 = _sc_layer(x0.reshape(2 * NP, H), src2, dst2, norm_f)
    taps1 = [x0] + [t.reshape(2, NP, H) for t in t1]
    x1 = _tc_combine(taps1, W1, b1.reshape(1, C))
    t2 = _sc_layer(x1.reshape(2 * NP, H), src2, dst2, norm_f)
    taps2 = [x1] + [t.reshape(2, NP, H) for t in t2]
    mu, sg = _tc_final(taps2, W2, b2.reshape(1, C),
                       Wout, bout.reshape(1, C))
    return mu[:N], sg[:N]


# speedup vs baseline: 1.6789x; 1.6789x over previous
import functools

import jax
import jax.numpy as jnp
from jax import lax
from jax.experimental import pallas as pl
from jax.experimental.pallas import tpu as pltpu
from jax.experimental.pallas import tpu_sc as plsc

N = 10000
E = 320000
D = 128
C = 32
H = 16
K = 4
NP = 10240
L = 16
NS = 16
NC = 2
CH = 128
NCHT = 160
TE = NCHT * CH
EP = NS * TE
EROWS = EP // CH
TSTR = NP // NS

_mesh = plsc.VectorSubcoreMesh(core_axis_name="c", subcore_axis_name="s")


def _iota16():
    return lax.iota(jnp.int32, L)


def _f16i(v):
    return jnp.full((L,), v, dtype=jnp.int32)


def _newton_rsqrt(d):
    i = plsc.bitcast(d, jnp.int32)
    y = plsc.bitcast(jnp.int32(0x5F3759DF) - lax.shift_right_arithmetic(i, 1),
                     jnp.float32)
    for _ in range(3):
        y = y * (1.5 - 0.5 * d * y * y)
    return y


TE2 = TE // 2


def _prep_body(src_in, dst_in, ea_in, norm_out,
               src_v, dst_v, ea_v, degl_v, acc_v, tmp_v, dis_v, dfull_v,
               nbuf_v, deg_sh, dis_sh):
    c = lax.axis_index("c")
    s = lax.axis_index("s")
    pltpu.sync_copy(src_in.at[pl.ds(s * TE, TE)], src_v)
    pltpu.sync_copy(dst_in.at[pl.ds(s * TE, TE)], dst_v)
    pltpu.sync_copy(ea_in.at[pl.ds(s * TE, TE)], ea_v)

    def _z(i, _):
        degl_v[pl.ds(i * L, L)] = jnp.zeros((L,), jnp.float32)
        return 0
    lax.fori_loop(0, NP // L, _z, 0)

    def _deg(i, _):
        d16 = dst_v[pl.ds(i * L, L)]
        e16 = ea_v[pl.ds(i * L, L)]
        plsc.addupdate_scatter(degl_v, [d16], e16)
        return 0
    lax.fori_loop(0, TE // L, _deg, 0)
    pltpu.sync_copy(degl_v, deg_sh.at[s])
    plsc.subcore_barrier()

    def _za(r, _):
        acc_v[pl.ds(r * L, L)] = jnp.zeros((L,), jnp.float32)
        return 0
    lax.fori_loop(0, TSTR // L, _za, 0)
    for u in range(NS):
        pltpu.sync_copy(deg_sh.at[u, pl.ds(s * TSTR, TSTR)], tmp_v)

        def _ra(r, _):
            acc_v[pl.ds(r * L, L)] = acc_v[pl.ds(r * L, L)] + \
                tmp_v[pl.ds(r * L, L)]
            return 0
        lax.fori_loop(0, TSTR // L, _ra, 0)

    def _dis(r, _):
        d = acc_v[pl.ds(r * L, L)]
        y = _newton_rsqrt(d + 1e-12)
        dis_v[pl.ds(r * L, L)] = jnp.where(d > 0.0, y, 0.0)
        return 0
    lax.fori_loop(0, TSTR // L, _dis, 0)
    pltpu.sync_copy(dis_v, dis_sh.at[pl.ds(s * TSTR, TSTR)])
    plsc.subcore_barrier()
    pltpu.sync_copy(dis_sh, dfull_v)

    def _nrm(i, _):
        o = c * TE2 + i * L
        s16 = src_v[pl.ds(o, L)]
        d16 = dst_v[pl.ds(o, L)]
        e16 = ea_v[pl.ds(o, L)]
        n16 = plsc.load_gather(dfull_v, [s16]) * e16 * \
            plsc.load_gather(dfull_v, [d16])
        nbuf_v[pl.ds(i * L, L)] = n16
        return 0
    lax.fori_loop(0, TE2 // L, _nrm, 0)
    pltpu.sync_copy(nbuf_v, norm_out.at[pl.ds(s * TE + c * TE2, TE2)])


_sc_prep = pl.kernel(
    _prep_body,
    out_type=jax.ShapeDtypeStruct((EP,), jnp.float32),
    mesh=_mesh,
    scratch_types=[
        pltpu.VMEM((TE,), jnp.int32),
        pltpu.VMEM((TE,), jnp.int32),
        pltpu.VMEM((TE,), jnp.float32),
        pltpu.VMEM((NP,), jnp.float32),
        pltpu.VMEM((TSTR,), jnp.float32),
        pltpu.VMEM((TSTR,), jnp.float32),
        pltpu.VMEM((TSTR,), jnp.float32),
        pltpu.VMEM((NP,), jnp.float32),
        pltpu.VMEM((TE2,), jnp.float32),
        pltpu.VMEM_SHARED((NS, NP), jnp.float32),
        pltpu.VMEM_SHARED((NP,), jnp.float32),
    ],
    compiler_params=pltpu.CompilerParams(
        needs_layout_passes=False, use_tc_tiling_on_sc=False),
    name="sc_prep",
)


NBUF = 4
NJO = NCHT // NBUF - 1


def _layer_body(x_in, src_in, dst_in, norm_in, to0, to1, to2, to3,
                src_v, dst_v, norm_v, g0, g1, g2, g3, b0, b1, b2, b3,
                zero_v, xa_sh, xb_sh, s0, s1, s2, s3, t0, t1, t2, t3):
    c = lax.axis_index("c")
    s = lax.axis_index("s")
    taps_out = (to0, to1, to2, to3)
    gbufs = (g0, g1, g2, g3)
    sbufs = (b0, b1, b2, b3)
    sems = (s0, s1, s2, s3)
    ssems = (t0, t1, t2, t3)
    r0 = s * NCHT
    pltpu.sync_copy(src_in.at[pl.ds(r0, NCHT)], src_v)
    pltpu.sync_copy(dst_in.at[pl.ds(r0, NCHT)], dst_v)
    pltpu.sync_copy(norm_in.at[pl.ds(s * TE, TE)], norm_v)

    def _z(i, _):
        zero_v[i] = jnp.zeros((L,), jnp.float32)
        return 0
    lax.fori_loop(0, TSTR, _z, 0)

    pltpu.sync_copy(x_in.at[pl.ds(c * NP + s * TSTR, TSTR)],
                    xa_sh.at[pl.ds(s * TSTR, TSTR)])
    pltpu.sync_copy(zero_v, xb_sh.at[pl.ds(s * TSTR, TSTR)])
    plsc.subcore_barrier()

    def _scale(j, b):
        def _g(g, _):
            n16 = norm_v[pl.ds(j * CH + g * L, L)]
            for e2 in range(L):
                spl = n16.at[_f16i(e2)].get(mode="promise_in_bounds")
                r = g * L + e2
                sbufs[b][r] = gbufs[b][r] * spl
            return 0
        lax.fori_loop(0, CH // L, _g, 0)

    def _chunk(j, b, xs, ys, wait_scat, issue_next):
        pltpu.make_async_copy(xs.at[src_v.at[j]], gbufs[b], sems[b]).wait()
        if wait_scat:
            pltpu.make_async_copy(sbufs[b], ys.at[dst_v.at[j]],
                                  ssems[b]).wait()
        _scale(j, b)
        pltpu.async_copy(sbufs[b], ys.at[dst_v.at[j]], ssems[b], add=True)
        if issue_next:
            pltpu.async_copy(xs.at[src_v.at[j + NBUF]], gbufs[b], sems[b])

    xs, ys = xa_sh, xb_sh
    for k in range(K):
        for b in range(NBUF):
            pltpu.async_copy(xs.at[src_v.at[b]], gbufs[b], sems[b])
        for b in range(NBUF):
            _chunk(b, b, xs, ys, False, True)

        def _pipe(jo, _, xs=xs, ys=ys):
            for b in range(NBUF):
                _chunk(jo * NBUF + b, b, xs, ys, True, True)
            return 0
        lax.fori_loop(1, NJO, _pipe, 0)
        for b in range(NBUF):
            _chunk(NJO * NBUF + b, b, xs, ys, True, False)
        for b in range(NBUF):
            pltpu.make_async_copy(sbufs[b], ys.at[dst_v.at[b]],
                                  ssems[b]).wait()
        plsc.subcore_barrier()
        pltpu.sync_copy(ys.at[pl.ds(s * TSTR, TSTR)],
                        taps_out[k].at[pl.ds(c * NP + s * TSTR, TSTR)])
        if k < K - 1:
            pltpu.sync_copy(zero_v, xs.at[pl.ds(s * TSTR, TSTR)])
            plsc.subcore_barrier()
        xs, ys = ys, xs


_sc_layer = pl.kernel(
    _layer_body,
    out_type=[jax.ShapeDtypeStruct((2 * NP, H), jnp.float32)] * K,
    mesh=_mesh,
    scratch_types=[
        pltpu.VMEM((NCHT, CH), jnp.int32),
        pltpu.VMEM((NCHT, CH), jnp.int32),
        pltpu.VMEM((TE,), jnp.float32),
        pltpu.VMEM((CH, H), jnp.float32),
        pltpu.VMEM((CH, H), jnp.float32),
        pltpu.VMEM((CH, H), jnp.float32),
        pltpu.VMEM((CH, H), jnp.float32),
        pltpu.VMEM((CH, H), jnp.float32),
        pltpu.VMEM((CH, H), jnp.float32),
        pltpu.VMEM((CH, H), jnp.float32),
        pltpu.VMEM((CH, H), jnp.float32),
        pltpu.VMEM((TSTR, H), jnp.float32),
        pltpu.VMEM_SHARED((NP, H), jnp.float32),
        pltpu.VMEM_SHARED((NP, H), jnp.float32),
        pltpu.SemaphoreType.DMA,
        pltpu.SemaphoreType.DMA,
        pltpu.SemaphoreType.DMA,
        pltpu.SemaphoreType.DMA,
        pltpu.SemaphoreType.DMA,
        pltpu.SemaphoreType.DMA,
        pltpu.SemaphoreType.DMA,
        pltpu.SemaphoreType.DMA,
    ],
    compiler_params=pltpu.CompilerParams(
        needs_layout_passes=False, use_tc_tiling_on_sc=False),
    name="sc_layer",
)


_BLK = 1024
_NBLK = NP // _BLK


def _leaky(t):
    return jnp.where(t >= 0.0, t, 0.01 * t)


def _pre_body(x_ref, w_ref, b_ref, o_ref):
    t = jnp.dot(x_ref[...], w_ref[...],
                preferred_element_type=jnp.float32) + b_ref[...]
    h = _leaky(t)
    o_ref[0] = h[:, :H]
    o_ref[1] = h[:, H:]


def _tc_pre(state_p, win, b_in):
    return pl.pallas_call(
        _pre_body,
        grid=(_NBLK,),
        in_specs=[
            pl.BlockSpec((_BLK, D), lambda i: (i, 0)),
            pl.BlockSpec((D, C), lambda i: (0, 0)),
            pl.BlockSpec((1, C), lambda i: (0, 0)),
        ],
        out_specs=pl.BlockSpec((2, _BLK, H), lambda i: (0, i, 0)),
        out_shape=jax.ShapeDtypeStruct((2, NP, H), jnp.float32),
    )(state_p, win, b_in)


def _cat(t):
    return jnp.concatenate([t[0], t[1]], axis=1)


def _acc_taps(taps, w_ref, b_ref):
    acc = b_ref[...]
    for k, t in enumerate(taps):
        acc = acc + jnp.dot(_cat(t[...]), w_ref[k],
                            preferred_element_type=jnp.float32)
    return acc


def _combine_body(t0, t1, t2, t3, t4, w_ref, b_ref, o_ref):
    h = _leaky(_acc_taps((t0, t1, t2, t3, t4), w_ref, b_ref))
    o_ref[0] = h[:, :H]
    o_ref[1] = h[:, H:]


def _tc_combine(taps, w, b):
    return pl.pallas_call(
        _combine_body,
        grid=(_NBLK,),
        in_specs=[pl.BlockSpec((2, _BLK, H), lambda i: (0, i, 0))] * 5 + [
            pl.BlockSpec((K + 1, C, C), lambda i: (0, 0, 0)),
            pl.BlockSpec((1, C), lambda i: (0, 0)),
        ],
        out_specs=pl.BlockSpec((2, _BLK, H), lambda i: (0, i, 0)),
        out_shape=jax.ShapeDtypeStruct((2, NP, H), jnp.float32),
    )(*taps, w, b)


def _final_body(t0, t1, t2, t3, t4, w_ref, b_ref, wo_ref, bo_ref,
                mu_ref, sg_ref):
    h = _leaky(_acc_taps((t0, t1, t2, t3, t4), w_ref, b_ref))
    o = jnp.dot(h, wo_ref[...], preferred_element_type=jnp.float32) + \
        bo_ref[...]
    mu_ref[...] = o[:, :H]
    o2 = o[:, H:]
    sg_ref[...] = jnp.maximum(o2, 0.0) + jnp.log1p(jnp.exp(-jnp.abs(o2)))


def _tc_final(taps, w, b, wo, bo):
    return pl.pallas_call(
        _final_body,
        grid=(_NBLK,),
        in_specs=[pl.BlockSpec((2, _BLK, H), lambda i: (0, i, 0))] * 5 + [
            pl.BlockSpec((K + 1, C, C), lambda i: (0, 0, 0)),
            pl.BlockSpec((1, C), lambda i: (0, 0)),
            pl.BlockSpec((C, C), lambda i: (0, 0)),
            pl.BlockSpec((1, C), lambda i: (0, 0)),
        ],
        out_specs=[pl.BlockSpec((_BLK, H), lambda i: (i, 0))] * 2,
        out_shape=[jax.ShapeDtypeStruct((NP, H), jnp.float32)] * 2,
    )(*taps, w, b, wo, bo)


def kernel(state, edge_index, edge_attr, Win, b_in, W1, b1, W2, b2,
           Wout, bout):
    src = edge_index[0].astype(jnp.int32)
    dst = edge_index[1].astype(jnp.int32)
    pad = EP - E
    src_f = jnp.pad(src, (0, pad))
    dst_f = jnp.pad(dst, (0, pad))
    src2 = src_f.reshape(EROWS, CH)
    dst2 = dst_f.reshape(EROWS, CH)
    ea_f = jnp.pad(edge_attr, (0, pad))
    state_p = jnp.pad(state, ((0, NP - N), (0, 0)))

    norm_f = _sc_prep(src_f, dst_f, ea_f)
    x0 = _tc_pre(state_p, Win, b_in.reshape(1, C))

    t1 = _sc_layer(x0.reshape(2 * NP, H), src2, dst2, norm_f)
    taps1 = [x0] + [t.reshape(2, NP, H) for t in t1]
    x1 = _tc_combine(taps1, W1, b1.reshape(1, C))
    t2 = _sc_layer(x1.reshape(2 * NP, H), src2, dst2, norm_f)
    taps2 = [x1] + [t.reshape(2, NP, H) for t in t2]
    mu, sg = _tc_final(taps2, W2, b2.reshape(1, C),
                       Wout, bout.reshape(1, C))
    return mu[:N], sg[:N]
